# Initial kernel scaffold; baseline (speedup 1.0000x reference)
import jax
import jax.numpy as jnp
from jax.experimental import pallas as pl


def kernel(actions, emb_tables, W, b):
    raise NotImplementedError("write your pallas kernel here")


# speedup vs baseline: 13.6666x; 13.6666x over previous
import functools

import jax
import jax.numpy as jnp
from jax import lax
from jax.experimental import pallas as pl
from jax.experimental.pallas import tpu as pltpu
from jax.experimental.pallas import tpu_sc as plsc

_ACTION_DIM = 10
_NUM_BINS = 256
_EMB = 12
_HID = 128
_TOK = _ACTION_DIM * _EMB


def _sc_gather(actions_t, table_flat, n_tokens):
    info = plsc.get_sparse_core_info()
    nc, ns, L = info.num_cores, info.num_subcores, info.num_lanes
    nw = nc * ns
    C = 256
    per_w = n_tokens // nw
    chunks = per_w // C
    mesh = plsc.VectorSubcoreMesh(core_axis_name="c", subcore_axis_name="s")

    @functools.partial(
        pl.kernel,
        mesh=mesh,
        out_type=jax.ShapeDtypeStruct((_TOK, n_tokens), jnp.float32),
        scratch_types=[
            pltpu.VMEM((_ACTION_DIM * _NUM_BINS * _EMB,), jnp.float32),
            pltpu.VMEM((_ACTION_DIM, C), jnp.float32),
            pltpu.VMEM((_TOK, C), jnp.float32),
        ],
        compiler_params=pltpu.CompilerParams(needs_layout_passes=False),
    )
    def k(actions_hbm, table_hbm, out_hbm, table_v, act_v, tok_v):
        wid = lax.axis_index("s") * nc + lax.axis_index("c")
        base = wid * per_w
        pltpu.sync_copy(table_hbm, table_v)

        def chunk_body(ci, carry):
            start = base + ci * C
            pltpu.sync_copy(actions_hbm.at[:, pl.ds(start, C)], act_v)

            def group(g, c2):
                off = g * L
                for d in range(_ACTION_DIM):
                    av = act_v[d, pl.ds(off, L)]
                    a = jnp.clip(av, -1.0, 1.0)
                    a = (a + 1.0) / 2.0 * 255.0
                    bins = a.astype(jnp.int32)
                    rowbase = bins * _EMB + d * (_NUM_BINS * _EMB)
                    for w in range(_EMB):
                        val = plsc.load_gather(table_v, [rowbase + w])
                        tok_v[d * _EMB + w, pl.ds(off, L)] = val
                return c2

            lax.fori_loop(0, C // L, group, 0)
            pltpu.sync_copy(tok_v, out_hbm.at[:, pl.ds(start, C)])
            return carry

        lax.fori_loop(0, chunks, chunk_body, 0)

    return k(actions_t, table_flat)


def _tc_project(tokens_t, w_bf16, b_row):
    n = tokens_t.shape[1]
    BT = 1024

    def mm(tok_ref, w_ref, b_ref, o_ref):
        x = tok_ref[...].astype(jnp.bfloat16)
        acc = lax.dot_general(
            x, w_ref[...], (((0,), (0,)), ((), ())),
            preferred_element_type=jnp.float32,
        )
        o_ref[...] = acc + b_ref[...]

    return pl.pallas_call(
        mm,
        grid=(n // BT,),
        in_specs=[
            pl.BlockSpec((_TOK, BT), lambda i: (0, i)),
            pl.BlockSpec((_TOK, _HID), lambda i: (0, 0)),
            pl.BlockSpec((1, _HID), lambda i: (0, 0)),
        ],
        out_specs=pl.BlockSpec((BT, _HID), lambda i: (i, 0)),
        out_shape=jax.ShapeDtypeStruct((n, _HID), jnp.float32),
    )(tokens_t, w_bf16, b_row)


def kernel(actions, emb_tables, W, b):
    bsz, t, d = actions.shape
    n = bsz * t
    actions_t = actions.reshape(n, d).T
    table_flat = emb_tables.reshape(-1)
    tokens_t = _sc_gather(actions_t, table_flat, n)
    out = _tc_project(tokens_t, W.astype(jnp.bfloat16), b.reshape(1, _HID))
    return out.reshape(bsz, t, _HID)


# speedup vs baseline: 20.9606x; 1.5337x over previous
import functools

import jax
import jax.numpy as jnp
from jax import lax
from jax.experimental import pallas as pl
from jax.experimental.pallas import tpu as pltpu
from jax.experimental.pallas import tpu_sc as plsc

_ACTION_DIM = 10
_NUM_BINS = 256
_EMB = 12
_HID = 128
_TOK = _ACTION_DIM * _EMB


def _sc_gather(actions_t, table_flat, n_tokens):
    info = plsc.get_sparse_core_info()
    nc, ns, L = info.num_cores, info.num_subcores, info.num_lanes
    nw = nc * ns
    C = 256
    per_w = n_tokens // nw
    chunks = per_w // C
    mesh = plsc.VectorSubcoreMesh(core_axis_name="c", subcore_axis_name="s")

    @functools.partial(
        pl.kernel,
        mesh=mesh,
        out_type=jax.ShapeDtypeStruct((_TOK, n_tokens), jnp.float32),
        scratch_types=[
            pltpu.VMEM((_ACTION_DIM * _NUM_BINS * _EMB,), jnp.float32),
            pltpu.VMEM((_ACTION_DIM, C), jnp.float32),
            pltpu.VMEM((_TOK, C), jnp.float32),
        ],
        compiler_params=pltpu.CompilerParams(needs_layout_passes=False),
    )
    def k(actions_hbm, table_hbm, out_hbm, table_v, act_v, tok_v):
        wid = lax.axis_index("s") * nc + lax.axis_index("c")
        base = wid * per_w
        pltpu.sync_copy(table_hbm, table_v)

        def chunk_body(ci, carry):
            start = base + ci * C
            pltpu.sync_copy(actions_hbm.at[:, pl.ds(start, C)], act_v)

            @plsc.parallel_loop(0, C // L, unroll=2)
            def group(g):
                off = g * L
                for d in range(_ACTION_DIM):
                    av = act_v[d, pl.ds(off, L)]
                    a = jnp.clip(av, -1.0, 1.0)
                    a = (a + 1.0) * 127.5
                    bins = a.astype(jnp.int32)
                    rowbase = bins * _EMB + d * (_NUM_BINS * _EMB)
                    for w in range(_EMB):
                        val = plsc.load_gather(table_v, [rowbase + w])
                        tok_v[d * _EMB + w, pl.ds(off, L)] = val
            pltpu.sync_copy(tok_v, out_hbm.at[:, pl.ds(start, C)])
            return carry

        lax.fori_loop(0, chunks, chunk_body, 0)

    return k(actions_t, table_flat)


def _tc_project(tokens_t, w_bf16, b_row):
    n = tokens_t.shape[1]
    BT = 4096

    def mm(tok_ref, w_ref, b_ref, o_ref):
        x = tok_ref[...].astype(jnp.bfloat16)
        acc = lax.dot_general(
            x, w_ref[...], (((0,), (0,)), ((), ())),
            preferred_element_type=jnp.float32,
        )
        o_ref[...] = acc + b_ref[...]

    return pl.pallas_call(
        mm,
        grid=(n // BT,),
        in_specs=[
            pl.BlockSpec((_TOK, BT), lambda i: (0, i)),
            pl.BlockSpec((_TOK, _HID), lambda i: (0, 0)),
            pl.BlockSpec((1, _HID), lambda i: (0, 0)),
        ],
        out_specs=pl.BlockSpec((BT, _HID), lambda i: (i, 0)),
        out_shape=jax.ShapeDtypeStruct((n, _HID), jnp.float32),
    )(tokens_t, w_bf16, b_row)


def kernel(actions, emb_tables, W, b):
    bsz, t, d = actions.shape
    n = bsz * t
    actions_t = actions.reshape(n, d).T
    table_flat = emb_tables.reshape(-1)
    tokens_t = _sc_gather(actions_t, table_flat, n)
    out = _tc_project(tokens_t, W.astype(jnp.bfloat16), b.reshape(1, _HID))
    return out.reshape(bsz, t, _HID)
